# baseline (device time: 45181 ns/iter reference)
import jax
import jax.numpy as jnp
from jax import lax
from jax.experimental import pallas as pl
from jax.experimental.pallas import tpu as pltpu

N_DEV = 4


def kernel(Q, K, V):
    b, q_len, h, d = Q.shape
    k_len = K.shape[1]
    scale = d ** -0.5

    def body(q_ref, k_ref, v_ref, out_ref, comm_ref, send_sems, recv_sems):
        my_pos = lax.axis_index("i")
        left = (my_pos - 1) % N_DEV
        right = (my_pos + 1) % N_DEV

        barrier_sem = pltpu.get_barrier_semaphore()
        for nbr in [left, right]:
            pl.semaphore_signal(
                barrier_sem, inc=1,
                device_id=(nbr,), device_id_type=pl.DeviceIdType.MESH,
            )
        pl.semaphore_wait(barrier_sem, 2)

        q = q_ref[:, 0, :, :]
        s = jnp.sum(k_ref[...] * q[:, None, :, :], axis=-1) * scale
        m = jnp.max(s, axis=1)
        p = jnp.exp(s - m[:, None, :])
        l = jnp.sum(p, axis=1)
        acc = jnp.sum(p[:, :, :, None] * v_ref[...], axis=1)

        comm_ref[0, 0:b] = acc
        comm_ref[0, b, :, 0:h] = m
        comm_ref[0, b, :, h:2 * h] = l

        m_r, l_r, acc_r = m, l, acc

        for t in range(N_DEV - 1):
            send_slot = t % 2
            recv_slot = (t + 1) % 2
            rdma = pltpu.make_async_remote_copy(
                src_ref=comm_ref.at[send_slot],
                dst_ref=comm_ref.at[recv_slot],
                send_sem=send_sems.at[send_slot],
                recv_sem=recv_sems.at[recv_slot],
                device_id=(right,),
                device_id_type=pl.DeviceIdType.MESH,
            )
            rdma.start()
            rdma.wait()

            acc_in = comm_ref[recv_slot, 0:b]
            m_in = comm_ref[recv_slot, b, :, 0:h]
            l_in = comm_ref[recv_slot, b, :, h:2 * h]

            m_new = jnp.maximum(m_r, m_in)
            w_r = jnp.exp(m_r - m_new)
            w_in = jnp.exp(m_in - m_new)
            l_r = w_r * l_r + w_in * l_in
            acc_r = w_r[:, :, None] * acc_r + w_in[:, :, None] * acc_in
            m_r = m_new

        out_ref[:, 0, :, :] = acc_r / l_r[:, :, None]

    return pl.pallas_call(
        body,
        out_shape=jax.ShapeDtypeStruct((b, q_len, h, d), jnp.float32),
        in_specs=[
            pl.BlockSpec(memory_space=pltpu.VMEM),
            pl.BlockSpec(memory_space=pltpu.VMEM),
            pl.BlockSpec(memory_space=pltpu.VMEM),
        ],
        out_specs=pl.BlockSpec(memory_space=pltpu.VMEM),
        scratch_shapes=[
            pltpu.VMEM((2, b + 1, h, d), jnp.float32),
            pltpu.SemaphoreType.DMA((2,)),
            pltpu.SemaphoreType.DMA((2,)),
        ],
        compiler_params=pltpu.CompilerParams(collective_id=0),
    )(Q, K, V)


# device time: 23977 ns/iter; 1.8843x vs baseline; 1.8843x over previous
import jax
import jax.numpy as jnp
from jax import lax
from jax.experimental import pallas as pl
from jax.experimental.pallas import tpu as pltpu

N_DEV = 4


def kernel(Q, K, V):
    b, q_len, h, d = Q.shape
    kk = K.shape[1]
    hd = h * d
    scale = d ** -0.5
    W = hd + 2 * h

    K2 = K.reshape(b, kk, hd)
    V2 = V.reshape(b, kk, hd)
    Qc = Q.reshape(b, hd, 1)

    def body(q_ref, k_ref, v_ref, out_ref, mine_ref, comm_ref,
             send_sems, recv_sems, ack_sem):
        my_pos = lax.axis_index("i")

        barrier_sem = pltpu.get_barrier_semaphore()
        for j in range(1, N_DEV):
            pl.semaphore_signal(
                barrier_sem, inc=1,
                device_id=((my_pos + j) % N_DEV,),
                device_id_type=pl.DeviceIdType.MESH,
            )

        qmask = (
            lax.broadcasted_iota(jnp.int32, (hd, h), 0) // d
            == lax.broadcasted_iota(jnp.int32, (hd, h), 1)
        )
        expand = (
            lax.broadcasted_iota(jnp.int32, (h, hd), 0)
            == lax.broadcasted_iota(jnp.int32, (h, hd), 1) // d
        ).astype(jnp.float32)

        ms, ls, accs = [], [], []
        for bi in range(b):
            qb = jnp.where(qmask, q_ref[bi], 0.0)
            s = jnp.dot(k_ref[bi], qb,
                        preferred_element_type=jnp.float32) * scale
            m_b = jnp.max(s, axis=0, keepdims=True)
            p = jnp.exp(s - m_b)
            l_b = jnp.sum(p, axis=0, keepdims=True)
            pexp = jnp.dot(p, expand,
                           preferred_element_type=jnp.float32)
            acc_b = jnp.sum(pexp * v_ref[bi], axis=0, keepdims=True)
            ms.append(m_b)
            ls.append(l_b)
            accs.append(acc_b)

        m_r = jnp.concatenate(ms, axis=0)
        l_r = jnp.concatenate(ls, axis=0)
        acc_r = jnp.concatenate(accs, axis=0)

        mine_ref[:, 0:hd] = acc_r
        mine_ref[:, hd:hd + h] = m_r
        mine_ref[:, hd + h:W] = l_r

        pl.semaphore_wait(barrier_sem, N_DEV - 1)

        sends = []
        for j in range(N_DEV - 1):
            rdma = pltpu.make_async_remote_copy(
                src_ref=mine_ref,
                dst_ref=comm_ref.at[2 - j],
                send_sem=send_sems.at[j],
                recv_sem=recv_sems.at[2 - j],
                device_id=((my_pos + 1 + j) % N_DEV,),
                device_id_type=pl.DeviceIdType.MESH,
            )
            rdma.start()
            sends.append(rdma)

        for slot in range(N_DEV - 1):
            recv = pltpu.make_async_remote_copy(
                src_ref=mine_ref,
                dst_ref=comm_ref.at[slot],
                send_sem=send_sems.at[0],
                recv_sem=recv_sems.at[slot],
                device_id=(my_pos,),
                device_id_type=pl.DeviceIdType.MESH,
            )
            recv.wait_recv()
            acc_in = comm_ref[slot, :, 0:hd]
            m_in = comm_ref[slot, :, hd:hd + h]
            l_in = comm_ref[slot, :, hd + h:W]

            m_new = jnp.maximum(m_r, m_in)
            w_r = jnp.exp(m_r - m_new)
            w_in = jnp.exp(m_in - m_new)
            l_r = w_r * l_r + w_in * l_in
            acc_r = (jnp.dot(w_r, expand) * acc_r
                     + jnp.dot(w_in, expand) * acc_in)
            m_r = m_new

        for j in range(1, N_DEV):
            pl.semaphore_signal(
                ack_sem, inc=1,
                device_id=((my_pos + j) % N_DEV,),
                device_id_type=pl.DeviceIdType.MESH,
            )
        for rdma in sends:
            rdma.wait_send()
        pl.semaphore_wait(ack_sem, N_DEV - 1)

        out_ref[...] = acc_r / jnp.dot(l_r, expand)

    out2 = pl.pallas_call(
        body,
        out_shape=jax.ShapeDtypeStruct((b, hd), jnp.float32),
        in_specs=[
            pl.BlockSpec(memory_space=pltpu.VMEM),
            pl.BlockSpec(memory_space=pltpu.VMEM),
            pl.BlockSpec(memory_space=pltpu.VMEM),
        ],
        out_specs=pl.BlockSpec(memory_space=pltpu.VMEM),
        scratch_shapes=[
            pltpu.VMEM((b, W), jnp.float32),
            pltpu.VMEM((N_DEV - 1, b, W), jnp.float32),
            pltpu.SemaphoreType.DMA((N_DEV - 1,)),
            pltpu.SemaphoreType.DMA((N_DEV - 1,)),
            pltpu.SemaphoreType.REGULAR,
        ],
        compiler_params=pltpu.CompilerParams(collective_id=0),
    )(Qc, K2, V2)
    return out2.reshape(b, q_len, h, d)


# device time: 22541 ns/iter; 2.0044x vs baseline; 1.0637x over previous
import jax
import jax.numpy as jnp
from jax import lax
from jax.experimental import pallas as pl
from jax.experimental.pallas import tpu as pltpu

N_DEV = 4


def kernel(Q, K, V):
    b, q_len, h, d = Q.shape
    kk = K.shape[1]
    hd = h * d
    scale = d ** -0.5
    W = hd + 2 * h

    Kt = K.transpose(0, 2, 3, 1).reshape(b, hd, kk)
    Vt = V.transpose(0, 2, 3, 1).reshape(b, hd, kk)
    Q2 = Q.reshape(b * h, d)

    def body(q_ref, k_ref, v_ref, out_ref, mine_ref, comm_ref,
             send_sems, recv_sems):
        my_pos = lax.axis_index("i")

        barrier_sem = pltpu.get_barrier_semaphore()
        for j in range(1, N_DEV):
            pl.semaphore_signal(
                barrier_sem, inc=1,
                device_id=((my_pos + j) % N_DEV,),
                device_id_type=pl.DeviceIdType.MESH,
            )

        e2 = (
            lax.broadcasted_iota(jnp.int32, (hd, h), 0) // d
            == lax.broadcasted_iota(jnp.int32, (hd, h), 1)
        ).astype(jnp.float32)
        t3 = jnp.dot(
            q_ref[...],
            (lax.broadcasted_iota(jnp.int32, (d, hd), 0)
             == lax.broadcasted_iota(jnp.int32, (d, hd), 1) % d
             ).astype(jnp.float32),
            preferred_element_type=jnp.float32,
        )
        mask_ht = (
            lax.broadcasted_iota(jnp.int32, (b * h, hd), 0) % h
            == lax.broadcasted_iota(jnp.int32, (b * h, hd), 1) // d
        )
        qbt_all = jnp.where(mask_ht, t3, 0.0) * scale

        for bi in range(b):
            st = jnp.dot(qbt_all[bi * h:(bi + 1) * h, :], k_ref[bi],
                         preferred_element_type=jnp.float32)
            m_b = jnp.max(st, axis=1, keepdims=True)
            p = jnp.exp(st - m_b)
            l_b = jnp.sum(p, axis=1, keepdims=True)
            pexp = jnp.dot(e2, p,
                           preferred_element_type=jnp.float32)
            acc_b = jnp.sum(pexp * v_ref[bi], axis=1, keepdims=True)
            mine_ref[0:hd, bi:bi + 1] = acc_b
            mine_ref[hd:hd + h, bi:bi + 1] = m_b
            mine_ref[hd + h:W, bi:bi + 1] = l_b

        pl.semaphore_wait(barrier_sem, N_DEV - 1)

        sends = []
        for j in range(N_DEV - 1):
            rdma = pltpu.make_async_remote_copy(
                src_ref=mine_ref,
                dst_ref=comm_ref.at[2 - j],
                send_sem=send_sems.at[j],
                recv_sem=recv_sems.at[2 - j],
                device_id=((my_pos + 1 + j) % N_DEV,),
                device_id_type=pl.DeviceIdType.MESH,
            )
            rdma.start()
            sends.append(rdma)

        acc_r = mine_ref[0:hd, :]
        m_r = mine_ref[hd:hd + h, :]
        l_r = mine_ref[hd + h:W, :]

        for slot in range(N_DEV - 1):
            recv = pltpu.make_async_remote_copy(
                src_ref=mine_ref,
                dst_ref=comm_ref.at[slot],
                send_sem=send_sems.at[0],
                recv_sem=recv_sems.at[slot],
                device_id=(my_pos,),
                device_id_type=pl.DeviceIdType.MESH,
            )
            recv.wait_recv()
            acc_in = comm_ref[slot, 0:hd, :]
            m_in = comm_ref[slot, hd:hd + h, :]
            l_in = comm_ref[slot, hd + h:W, :]

            m_new = jnp.maximum(m_r, m_in)
            w_r = jnp.exp(m_r - m_new)
            w_in = jnp.exp(m_in - m_new)
            l_r = w_r * l_r + w_in * l_in
            acc_r = (jnp.dot(e2, w_r, preferred_element_type=jnp.float32)
                     * acc_r
                     + jnp.dot(e2, w_in, preferred_element_type=jnp.float32)
                     * acc_in)
            m_r = m_new

        for rdma in sends:
            rdma.wait_send()

        out = acc_r / jnp.dot(e2, l_r, preferred_element_type=jnp.float32)
        out_ref[...] = jnp.swapaxes(out, 0, 1)

    out2 = pl.pallas_call(
        body,
        out_shape=jax.ShapeDtypeStruct((b, hd), jnp.float32),
        in_specs=[
            pl.BlockSpec(memory_space=pltpu.VMEM),
            pl.BlockSpec(memory_space=pltpu.VMEM),
            pl.BlockSpec(memory_space=pltpu.VMEM),
        ],
        out_specs=pl.BlockSpec(memory_space=pltpu.VMEM),
        scratch_shapes=[
            pltpu.VMEM((W, b), jnp.float32),
            pltpu.VMEM((N_DEV - 1, W, b), jnp.float32),
            pltpu.SemaphoreType.DMA((N_DEV - 1,)),
            pltpu.SemaphoreType.DMA((N_DEV - 1,)),
        ],
        compiler_params=pltpu.CompilerParams(collective_id=0),
    )(Q2, Kt, Vt)
    return out2.reshape(b, q_len, h, d)
